# Initial kernel scaffold; baseline (speedup 1.0000x reference)
import jax
import jax.numpy as jnp
from jax.experimental import pallas as pl


def kernel(x, edge_index, batch, W1, a_src1, a_dst1, b1, W2, a_src2, a_dst2, b2, fc_w, fc_b):
    raise NotImplementedError("write your pallas kernel here")


# speedup vs baseline: 4.2243x; 4.2243x over previous
import functools

import jax
import jax.numpy as jnp
from jax import lax
from jax.experimental import pallas as pl
from jax.experimental.pallas import tpu as pltpu

N_NODES = 10000
N_EDGES = 160000
HEADS = 8
HID = 64
GROUPS = 64


def _t1_body(x_ref, w_ref, a_ref, h_ref, al_ref):
    h = jnp.dot(x_ref[...], w_ref[...], preferred_element_type=jnp.float32)
    h_ref[...] = h
    al_ref[...] = jnp.dot(h, a_ref[...], preferred_element_type=jnp.float32)


def _t1(x, W1, A1):
    n, d = x.shape
    dh = W1.shape[1]
    blk = 1000
    grid = n // blk
    return pl.pallas_call(
        _t1_body,
        grid=(grid,),
        in_specs=[
            pl.BlockSpec((blk, d), lambda i: (i, 0)),
            pl.BlockSpec((d, dh), lambda i: (0, 0)),
            pl.BlockSpec((dh, 2 * HEADS), lambda i: (0, 0)),
        ],
        out_specs=[
            pl.BlockSpec((blk, dh), lambda i: (i, 0)),
            pl.BlockSpec((blk, 2 * HEADS), lambda i: (i, 0)),
        ],
        out_shape=[
            jax.ShapeDtypeStruct((n, dh), jnp.float32),
            jax.ShapeDtypeStruct((n, 2 * HEADS), jnp.float32),
        ],
    )(x, W1, A1)


def _t2_body(acc_ref, den_ref, b1_ref, w2_ref, a2_ref, h2_ref, al2_ref):
    r = 1.0 / (den_ref[...] + 1e-16)
    heads_of_col = lax.broadcasted_iota(jnp.int32, (HEADS, HEADS * HID), 1) // HID
    e8 = (lax.broadcasted_iota(jnp.int32, (HEADS, HEADS * HID), 0)
          == heads_of_col).astype(jnp.float32)
    rexp = jnp.dot(r, e8, preferred_element_type=jnp.float32)
    h1n = jnp.maximum(acc_ref[...] * rexp + b1_ref[...], 0.0)
    h2 = jnp.dot(h1n, w2_ref[...], preferred_element_type=jnp.float32)
    h2_ref[...] = h2
    al2_ref[...] = jnp.dot(h2, a2_ref[...], preferred_element_type=jnp.float32)


def _t2(acc1, den1, b1, W2, A2):
    n, dh = acc1.shape
    blk = 1000
    grid = n // blk
    return pl.pallas_call(
        _t2_body,
        grid=(grid,),
        in_specs=[
            pl.BlockSpec((blk, dh), lambda i: (i, 0)),
            pl.BlockSpec((blk, HEADS), lambda i: (i, 0)),
            pl.BlockSpec((1, dh), lambda i: (0, 0)),
            pl.BlockSpec((dh, HID), lambda i: (0, 0)),
            pl.BlockSpec((HID, 16), lambda i: (0, 0)),
        ],
        out_specs=[
            pl.BlockSpec((blk, HID), lambda i: (i, 0)),
            pl.BlockSpec((blk, 16), lambda i: (i, 0)),
        ],
        out_shape=[
            jax.ShapeDtypeStruct((n, HID), jnp.float32),
            jax.ShapeDtypeStruct((n, 16), jnp.float32),
        ],
    )(acc1, den1, b1, W2, A2)


def _t3_body(acc2_ref, den8_ref, b2_ref, batch_ref, fcw_ref, fcb_ref, out_ref):
    n = acc2_ref.shape[0]
    r = 1.0 / (den8_ref[:, 0:1] + 1e-16)
    h2f = acc2_ref[...] * r + b2_ref[...]
    g = lax.broadcasted_iota(jnp.int32, (GROUPS, n), 0)
    onehot = (batch_ref[...] == g).astype(jnp.float32)
    pooled = jnp.dot(onehot, h2f, preferred_element_type=jnp.float32)
    logits = jnp.dot(pooled, fcw_ref[...],
                     preferred_element_type=jnp.float32) + fcb_ref[...]
    m = jnp.max(logits, axis=1, keepdims=True)
    s = logits - m
    out_ref[...] = s - jnp.log(jnp.sum(jnp.exp(s), axis=1, keepdims=True))


def _t3(acc2, den8, b2, batch2d, fc_w, fc_b):
    n = acc2.shape[0]
    c = fc_w.shape[1]
    return pl.pallas_call(
        _t3_body,
        out_shape=jax.ShapeDtypeStruct((GROUPS, c), jnp.float32),
    )(acc2, den8, b2, batch2d, fc_w, fc_b)


def _leaky(x):
    return jnp.maximum(x, 0.2 * x)


def kernel(x, edge_index, batch, W1, a_src1, a_dst1, b1, W2, a_src2, a_dst2,
           b2, fc_w, fc_b):
    src, dst = edge_index[0], edge_index[1]
    eye8 = jnp.eye(HEADS, dtype=jnp.float32)
    A_s1 = (a_src1[:, :, None] * eye8[:, None, :]).reshape(HEADS * HID, HEADS)
    A_d1 = (a_dst1[:, :, None] * eye8[:, None, :]).reshape(HEADS * HID, HEADS)
    A1 = jnp.concatenate([A_s1, A_d1], axis=1)
    A2 = jnp.concatenate(
        [a_src2.T, a_dst2.T, jnp.zeros((HID, 14), jnp.float32)], axis=1)

    h1, al1 = _t1(x, W1, A1)
    asrc1, adst1 = al1[:, :HEADS], al1[:, HEADS:]

    alpha = _leaky(asrc1[src] + adst1[dst])
    ex = jnp.exp(alpha)
    den1 = jax.ops.segment_sum(ex, dst, num_segments=N_NODES)
    msg = h1[src] * jnp.repeat(ex, HID, axis=1)
    acc1 = jax.ops.segment_sum(msg, dst, num_segments=N_NODES)

    h2, al2 = _t2(acc1, den1, b1[None, :], W2, A2)
    as2, ad2 = al2[:, 0], al2[:, 1]

    alpha2 = _leaky(as2[src] + ad2[dst])
    ex2 = jnp.exp(alpha2)
    den2 = jax.ops.segment_sum(ex2, dst, num_segments=N_NODES)
    acc2 = jax.ops.segment_sum(h2[src] * ex2[:, None], dst,
                               num_segments=N_NODES)

    den8 = jnp.pad(den2[:, None], ((0, 0), (0, 7)))
    return _t3(acc2, den8, b2[None, :], batch[None, :], fc_w, fc_b[None, :])


# speedup vs baseline: 8.8828x; 2.1028x over previous
import jax
import jax.numpy as jnp
from jax import lax
from jax.experimental import pallas as pl
from jax.experimental.pallas import tpu as pltpu
import jax.experimental.pallas.tpu_sc as plsc

N_NODES = 10000
NP = 10240
N_EDGES = 160000
HEADS = 8
HID = 64
DH = HEADS * HID
GROUPS = 64
L = 16

NC = 2
NS = 16
NW = NC * NS
WIN = 24
EP = 161280
EW = EP // NW
NWIN = EW // WIN
ROWS_T = NP // NS
FCH = 128
NCH = DH // FCH


def _sc1_body(tab_s, tab_d, src, dst, z16, z128, h0, h1, h2, h3,
              den_out, acc_out,
              srcw, dstw, rows_s, rows_d, rows,
              tabs_sh, tabd_sh, den_acc, acc):
    c = lax.axis_index("c")
    s = lax.axis_index("s")
    w = c * NS + s
    ebase = w * EW
    r0 = s * ROWS_T

    pltpu.sync_copy(tab_s.at[pl.ds(r0, ROWS_T)], tabs_sh.at[pl.ds(r0, ROWS_T)])
    pltpu.sync_copy(tab_d.at[pl.ds(r0, ROWS_T)], tabd_sh.at[pl.ds(r0, ROWS_T)])
    pltpu.sync_copy(z16.at[pl.ds(r0, ROWS_T)], den_acc.at[pl.ds(r0, ROWS_T)])
    plsc.subcore_barrier()

    def p1(j, carry):
        eo = pl.multiple_of(j * WIN, 8)
        pltpu.sync_copy(src.at[pl.ds(ebase + eo, WIN)], srcw)
        pltpu.sync_copy(dst.at[pl.ds(ebase + eo, WIN)], dstw)
        pltpu.sync_copy(tabs_sh.at[srcw], rows_s)
        pltpu.sync_copy(tabd_sh.at[dstw], rows_d)
        for e in range(WIN):
            a = rows_s[e, :] + rows_d[e, :]
            a = jnp.maximum(a, 0.2 * a)
            rows_s[e, :] = jnp.exp(a)
        pltpu.sync_copy(rows_s, den_acc.at[dstw], add=True)
        return carry

    lax.fori_loop(0, NWIN, p1, 0)
    plsc.subcore_barrier()
    pltpu.sync_copy(den_acc.at[pl.ds(r0, ROWS_T)],
                    den_out.at[c, pl.ds(r0, ROWS_T)])

    for ch, tab in enumerate((h0, h1, h2, h3)):
        pltpu.sync_copy(z128.at[pl.ds(r0, ROWS_T)], acc.at[pl.ds(r0, ROWS_T)])
        plsc.subcore_barrier()

        def p2(j, carry):
            eo = pl.multiple_of(j * WIN, 8)
            pltpu.sync_copy(src.at[pl.ds(ebase + eo, WIN)], srcw)
            pltpu.sync_copy(dst.at[pl.ds(ebase + eo, WIN)], dstw)
            pltpu.sync_copy(tabs_sh.at[srcw], rows_s)
            pltpu.sync_copy(tabd_sh.at[dstw], rows_d)
            pltpu.sync_copy(tab.at[srcw], rows)
            for e in range(WIN):
                a = rows_s[e, :] + rows_d[e, :]
                a = jnp.maximum(a, 0.2 * a)
                v = jnp.exp(a)
                s0 = v[2 * ch]
                s1 = v[2 * ch + 1]
                for k in range(4):
                    rows[e, pl.ds(k * L, L)] = rows[e, pl.ds(k * L, L)] * s0
                for k in range(4, 8):
                    rows[e, pl.ds(k * L, L)] = rows[e, pl.ds(k * L, L)] * s1
            pltpu.sync_copy(rows, acc.at[dstw], add=True)
            return carry

        lax.fori_loop(0, NWIN, p2, 0)
        plsc.subcore_barrier()
        pltpu.sync_copy(acc.at[pl.ds(r0, ROWS_T)],
                        acc_out.at[c, pl.ds(r0, ROWS_T), pl.ds(ch * FCH, FCH)])
        plsc.subcore_barrier()


def _sc1():
    f32 = jnp.float32
    mesh = plsc.VectorSubcoreMesh(core_axis_name="c", subcore_axis_name="s",
                                  num_cores=NC, num_subcores=NS)
    return pl.kernel(
        _sc1_body,
        out_type=[
            jax.ShapeDtypeStruct((NC, NP, L), f32),
            jax.ShapeDtypeStruct((NC, NP, DH), f32),
        ],
        mesh=mesh,
        scratch_types=[
            pltpu.VMEM((WIN,), jnp.int32),
            pltpu.VMEM((WIN,), jnp.int32),
            pltpu.VMEM((WIN, L), f32),
            pltpu.VMEM((WIN, L), f32),
            pltpu.VMEM((WIN, FCH), f32),
            pltpu.VMEM_SHARED((NP, L), f32),
            pltpu.VMEM_SHARED((NP, L), f32),
            pltpu.VMEM_SHARED((NP, L), f32),
            pltpu.VMEM_SHARED((NP, FCH), f32),
        ],
    )


def _sc2_body(tab_s, tab_d, src, dst, z16, z64, hfeat,
              den_out, acc_out,
              srcw, dstw, rows_s, rows_d, exw, rows, msg,
              tabs_sh, tabd_sh, den_acc, acc):
    c = lax.axis_index("c")
    s = lax.axis_index("s")
    w = c * NS + s
    ebase = w * EW
    r0 = s * ROWS_T

    pltpu.sync_copy(tab_s.at[pl.ds(r0, ROWS_T)], tabs_sh.at[pl.ds(r0, ROWS_T)])
    pltpu.sync_copy(tab_d.at[pl.ds(r0, ROWS_T)], tabd_sh.at[pl.ds(r0, ROWS_T)])
    pltpu.sync_copy(z16.at[pl.ds(r0, ROWS_T)], den_acc.at[pl.ds(r0, ROWS_T)])
    pltpu.sync_copy(z64.at[pl.ds(r0, ROWS_T)], acc.at[pl.ds(r0, ROWS_T)])
    plsc.subcore_barrier()

    def p(j, carry):
        eo = pl.multiple_of(j * WIN, 8)
        pltpu.sync_copy(src.at[pl.ds(ebase + eo, WIN)], srcw)
        pltpu.sync_copy(dst.at[pl.ds(ebase + eo, WIN)], dstw)
        pltpu.sync_copy(tabs_sh.at[srcw], rows_s)
        pltpu.sync_copy(tabd_sh.at[dstw], rows_d)
        pltpu.sync_copy(hfeat.at[srcw], rows)
        for e in range(WIN):
            a = rows_s[e, :] + rows_d[e, :]
            a = jnp.maximum(a, 0.2 * a)
            v = jnp.exp(a)
            exw[e, :] = v
            s0 = v[0]
            for k in range(4):
                msg[e, pl.ds(k * L, L)] = rows[e, pl.ds(k * L, L)] * s0
        pltpu.sync_copy(exw, den_acc.at[dstw], add=True)
        pltpu.sync_copy(msg, acc.at[dstw], add=True)
        return carry

    lax.fori_loop(0, NWIN, p, 0)
    plsc.subcore_barrier()
    pltpu.sync_copy(den_acc.at[pl.ds(r0, ROWS_T)],
                    den_out.at[c, pl.ds(r0, ROWS_T)])
    pltpu.sync_copy(acc.at[pl.ds(r0, ROWS_T)],
                    acc_out.at[c, pl.ds(r0, ROWS_T)])


def _sc2():
    f32 = jnp.float32
    mesh = plsc.VectorSubcoreMesh(core_axis_name="c", subcore_axis_name="s",
                                  num_cores=NC, num_subcores=NS)
    return pl.kernel(
        _sc2_body,
        out_type=[
            jax.ShapeDtypeStruct((NC, NP, L), f32),
            jax.ShapeDtypeStruct((NC, NP, HID), f32),
        ],
        mesh=mesh,
        scratch_types=[
            pltpu.VMEM((WIN,), jnp.int32),
            pltpu.VMEM((WIN,), jnp.int32),
            pltpu.VMEM((WIN, L), f32),
            pltpu.VMEM((WIN, L), f32),
            pltpu.VMEM((WIN, L), f32),
            pltpu.VMEM((WIN, FCH), f32),
            pltpu.VMEM((WIN, HID), f32),
            pltpu.VMEM_SHARED((NP, L), f32),
            pltpu.VMEM_SHARED((NP, L), f32),
            pltpu.VMEM_SHARED((NP, L), f32),
            pltpu.VMEM_SHARED((NP, HID), f32),
        ],
    )


def _t1_body(x_ref, w_ref, as_ref, ad_ref,
             h0_ref, h1_ref, h2_ref, h3_ref, als_ref, ald_ref):
    h = jnp.dot(x_ref[...], w_ref[...], preferred_element_type=jnp.float32)
    h0_ref[...] = h[:, 0 * FCH:1 * FCH]
    h1_ref[...] = h[:, 1 * FCH:2 * FCH]
    h2_ref[...] = h[:, 2 * FCH:3 * FCH]
    h3_ref[...] = h[:, 3 * FCH:4 * FCH]
    als_ref[...] = jnp.dot(h, as_ref[...], preferred_element_type=jnp.float32)
    ald_ref[...] = jnp.dot(h, ad_ref[...], preferred_element_type=jnp.float32)


def _t1(x, W1, A_s, A_d):
    n, d = x.shape
    blk = 1024
    grid = n // blk
    return pl.pallas_call(
        _t1_body,
        grid=(grid,),
        in_specs=[
            pl.BlockSpec((blk, d), lambda i: (i, 0)),
            pl.BlockSpec((d, DH), lambda i: (0, 0)),
            pl.BlockSpec((DH, L), lambda i: (0, 0)),
            pl.BlockSpec((DH, L), lambda i: (0, 0)),
        ],
        out_specs=[pl.BlockSpec((blk, FCH), lambda i: (i, 0))] * NCH
        + [pl.BlockSpec((blk, L), lambda i: (i, 0))] * 2,
        out_shape=[jax.ShapeDtypeStruct((n, FCH), jnp.float32)] * NCH
        + [jax.ShapeDtypeStruct((n, L), jnp.float32)] * 2,
    )(x, W1, A_s, A_d)


def _t2_body(acc_ref, den_ref, b1_ref, w2_ref, a2s_ref, a2d_ref,
             h2_ref, t2s_ref, t2d_ref):
    r = 1.0 / (den_ref[0, :, :HEADS] + den_ref[1, :, :HEADS] + 1e-16)
    heads_of_col = lax.broadcasted_iota(jnp.int32, (HEADS, DH), 1) // HID
    e8 = (lax.broadcasted_iota(jnp.int32, (HEADS, DH), 0)
          == heads_of_col).astype(jnp.float32)
    rexp = jnp.dot(r, e8, preferred_element_type=jnp.float32)
    acc = acc_ref[0] + acc_ref[1]
    h1n = jnp.maximum(acc * rexp + b1_ref[...], 0.0)
    h2 = jnp.dot(h1n, w2_ref[...], preferred_element_type=jnp.float32)
    h2_ref[...] = jnp.concatenate([h2, jnp.zeros_like(h2)], axis=1)
    t2s_ref[...] = jnp.dot(h2, a2s_ref[...], preferred_element_type=jnp.float32)
    t2d_ref[...] = jnp.dot(h2, a2d_ref[...], preferred_element_type=jnp.float32)


def _t2(acc1, den1, b1, W2, A2s, A2d):
    blk = 1024
    grid = NP // blk
    return pl.pallas_call(
        _t2_body,
        grid=(grid,),
        in_specs=[
            pl.BlockSpec((NC, blk, DH), lambda i: (0, i, 0)),
            pl.BlockSpec((NC, blk, L), lambda i: (0, i, 0)),
            pl.BlockSpec((1, DH), lambda i: (0, 0)),
            pl.BlockSpec((DH, HID), lambda i: (0, 0)),
            pl.BlockSpec((HID, L), lambda i: (0, 0)),
            pl.BlockSpec((HID, L), lambda i: (0, 0)),
        ],
        out_specs=[
            pl.BlockSpec((blk, FCH), lambda i: (i, 0)),
            pl.BlockSpec((blk, L), lambda i: (i, 0)),
            pl.BlockSpec((blk, L), lambda i: (i, 0)),
        ],
        out_shape=[
            jax.ShapeDtypeStruct((NP, FCH), jnp.float32),
            jax.ShapeDtypeStruct((NP, L), jnp.float32),
            jax.ShapeDtypeStruct((NP, L), jnp.float32),
        ],
    )(acc1, den1, b1, W2, A2s, A2d)


def _t3_body(acc_ref, den_ref, b2_ref, batch_ref, fcw_ref, fcb_ref, out_ref):
    n = acc_ref.shape[1]
    r = 1.0 / (den_ref[0, :, 0:1] + den_ref[1, :, 0:1] + 1e-16)
    h2f = (acc_ref[0] + acc_ref[1]) * r + b2_ref[...]
    g = lax.broadcasted_iota(jnp.int32, (GROUPS, n), 0)
    onehot = (batch_ref[...] == g).astype(jnp.float32)
    pooled = jnp.dot(onehot, h2f, preferred_element_type=jnp.float32)
    logits = jnp.dot(pooled, fcw_ref[...],
                     preferred_element_type=jnp.float32) + fcb_ref[...]
    m = jnp.max(logits, axis=1, keepdims=True)
    sh = logits - m
    out_ref[...] = sh - jnp.log(jnp.sum(jnp.exp(sh), axis=1, keepdims=True))


def _t3(acc2, den2, b2, batch2d, fc_w, fc_b):
    c = fc_w.shape[1]
    return pl.pallas_call(
        _t3_body,
        out_shape=jax.ShapeDtypeStruct((GROUPS, c), jnp.float32),
    )(acc2, den2, b2, batch2d, fc_w, fc_b)


def kernel(x, edge_index, batch, W1, a_src1, a_dst1, b1, W2, a_src2, a_dst2,
           b2, fc_w, fc_b):
    f32 = jnp.float32
    src = jnp.pad(edge_index[0], (0, EP - N_EDGES), constant_values=N_NODES)
    dst = jnp.pad(edge_index[1], (0, EP - N_EDGES), constant_values=N_NODES)

    eye8 = jnp.eye(HEADS, dtype=f32)
    A_s = jnp.pad((a_src1[:, :, None] * eye8[:, None, :]).reshape(DH, HEADS),
                  ((0, 0), (0, L - HEADS)))
    A_d = jnp.pad((a_dst1[:, :, None] * eye8[:, None, :]).reshape(DH, HEADS),
                  ((0, 0), (0, L - HEADS)))
    A2s = jnp.tile(a_src2.T, (1, L))
    A2d = jnp.tile(a_dst2.T, (1, L))

    x_pad = jnp.pad(x, ((0, NP - N_NODES), (0, 0)))
    h0, h1, h2c, h3, tab_s, tab_d = _t1(x_pad, W1, A_s, A_d)

    z16 = jnp.zeros((NP, L), f32)
    z128 = jnp.zeros((NP, FCH), f32)
    z64 = jnp.zeros((NP, HID), f32)

    den1, acc1 = _sc1()(tab_s, tab_d, src, dst, z16, z128, h0, h1, h2c, h3)

    h2, tab2s, tab2d = _t2(acc1, den1, b1[None, :], W2, A2s, A2d)

    den2, acc2 = _sc2()(tab2s, tab2d, src, dst, z16, z64, h2)

    batch_pad = jnp.pad(batch, (0, NP - N_NODES),
                        constant_values=GROUPS)[None, :]
    return _t3(acc2, den2, b2[None, :], batch_pad, fc_w, fc_b[None, :])


# speedup vs baseline: 9.6177x; 1.0827x over previous
import jax
import jax.numpy as jnp
from jax import lax
from jax.experimental import pallas as pl
from jax.experimental.pallas import tpu as pltpu
import jax.experimental.pallas.tpu_sc as plsc

N_NODES = 10000
NP = 10240
N_EDGES = 160000
HEADS = 8
HID = 64
DH = HEADS * HID
GROUPS = 64
L = 16

NC = 2
NS = 16
NW = NC * NS
WIN = 24
EP = 161280
EW = EP // NW
NWIN = EW // WIN
ROWS_T = NP // NS
FCH = 128
NCH = DH // FCH


def _sc1_body(tab_s, tab_d, src, dst, z16, z128, h0, h1, h2, h3,
              den_out, acc_out,
              srcw, dstw, rows_s, rows_d, exw, rows,
              tabs_sh, tabd_sh, den_acc, acc):
    c = lax.axis_index("c")
    s = lax.axis_index("s")
    w = c * NS + s
    ebase = w * EW
    r0 = s * ROWS_T

    pltpu.sync_copy(tab_s.at[pl.ds(r0, ROWS_T)], tabs_sh.at[pl.ds(r0, ROWS_T)])
    pltpu.sync_copy(tab_d.at[pl.ds(r0, ROWS_T)], tabd_sh.at[pl.ds(r0, ROWS_T)])
    pltpu.sync_copy(z16.at[pl.ds(r0, ROWS_T)], den_acc.at[pl.ds(r0, ROWS_T)])
    plsc.subcore_barrier()

    for ch, tab in enumerate((h0, h1, h2, h3)):
        pltpu.sync_copy(z128.at[pl.ds(r0, ROWS_T)], acc.at[pl.ds(r0, ROWS_T)])
        plsc.subcore_barrier()

        def p2(j, carry):
            eo = pl.multiple_of(j * WIN, 8)
            pltpu.sync_copy(src.at[pl.ds(ebase + eo, WIN)], srcw)
            pltpu.sync_copy(dst.at[pl.ds(ebase + eo, WIN)], dstw)
            pltpu.sync_copy(tabs_sh.at[srcw], rows_s)
            pltpu.sync_copy(tabd_sh.at[dstw], rows_d)
            pltpu.sync_copy(tab.at[srcw], rows)
            for e in range(WIN):
                a = rows_s[e, :] + rows_d[e, :]
                a = jnp.maximum(a, 0.2 * a)
                v = jnp.exp(a)
                if ch == 0:
                    exw[e, :] = v
                s0 = v[2 * ch]
                s1 = v[2 * ch + 1]
                for k in range(4):
                    rows[e, pl.ds(k * L, L)] = rows[e, pl.ds(k * L, L)] * s0
                for k in range(4, 8):
                    rows[e, pl.ds(k * L, L)] = rows[e, pl.ds(k * L, L)] * s1
            if ch == 0:
                pltpu.sync_copy(exw, den_acc.at[dstw], add=True)
            pltpu.sync_copy(rows, acc.at[dstw], add=True)
            return carry

        lax.fori_loop(0, NWIN, p2, 0)
        plsc.subcore_barrier()
        if ch == 0:
            pltpu.sync_copy(den_acc.at[pl.ds(r0, ROWS_T)],
                            den_out.at[c, pl.ds(r0, ROWS_T)])
        pltpu.sync_copy(acc.at[pl.ds(r0, ROWS_T)],
                        acc_out.at[c, pl.ds(r0, ROWS_T), pl.ds(ch * FCH, FCH)])
        plsc.subcore_barrier()


def _sc1():
    f32 = jnp.float32
    mesh = plsc.VectorSubcoreMesh(core_axis_name="c", subcore_axis_name="s",
                                  num_cores=NC, num_subcores=NS)
    return pl.kernel(
        _sc1_body,
        out_type=[
            jax.ShapeDtypeStruct((NC, NP, L), f32),
            jax.ShapeDtypeStruct((NC, NP, DH), f32),
        ],
        mesh=mesh,
        scratch_types=[
            pltpu.VMEM((WIN,), jnp.int32),
            pltpu.VMEM((WIN,), jnp.int32),
            pltpu.VMEM((WIN, L), f32),
            pltpu.VMEM((WIN, L), f32),
            pltpu.VMEM((WIN, L), f32),
            pltpu.VMEM((WIN, FCH), f32),
            pltpu.VMEM_SHARED((NP, L), f32),
            pltpu.VMEM_SHARED((NP, L), f32),
            pltpu.VMEM_SHARED((NP, L), f32),
            pltpu.VMEM_SHARED((NP, FCH), f32),
        ],
    )


def _sc2_body(tab_s, tab_d, src, dst, z16, z64, hfeat,
              den_out, acc_out,
              srcw, dstw, rows_s, rows_d, exw, rows, msg,
              tabs_sh, tabd_sh, den_acc, acc):
    c = lax.axis_index("c")
    s = lax.axis_index("s")
    w = c * NS + s
    ebase = w * EW
    r0 = s * ROWS_T

    pltpu.sync_copy(tab_s.at[pl.ds(r0, ROWS_T)], tabs_sh.at[pl.ds(r0, ROWS_T)])
    pltpu.sync_copy(tab_d.at[pl.ds(r0, ROWS_T)], tabd_sh.at[pl.ds(r0, ROWS_T)])
    pltpu.sync_copy(z16.at[pl.ds(r0, ROWS_T)], den_acc.at[pl.ds(r0, ROWS_T)])
    pltpu.sync_copy(z64.at[pl.ds(r0, ROWS_T)], acc.at[pl.ds(r0, ROWS_T)])
    plsc.subcore_barrier()

    def p(j, carry):
        eo = pl.multiple_of(j * WIN, 8)
        pltpu.sync_copy(src.at[pl.ds(ebase + eo, WIN)], srcw)
        pltpu.sync_copy(dst.at[pl.ds(ebase + eo, WIN)], dstw)
        pltpu.sync_copy(tabs_sh.at[srcw], rows_s)
        pltpu.sync_copy(tabd_sh.at[dstw], rows_d)
        pltpu.sync_copy(hfeat.at[srcw], rows)
        for e in range(WIN):
            a = rows_s[e, :] + rows_d[e, :]
            a = jnp.maximum(a, 0.2 * a)
            v = jnp.exp(a)
            exw[e, :] = v
            s0 = v[0]
            for k in range(4):
                msg[e, pl.ds(k * L, L)] = rows[e, pl.ds(k * L, L)] * s0
        pltpu.sync_copy(exw, den_acc.at[dstw], add=True)
        pltpu.sync_copy(msg, acc.at[dstw], add=True)
        return carry

    lax.fori_loop(0, NWIN, p, 0)
    plsc.subcore_barrier()
    pltpu.sync_copy(den_acc.at[pl.ds(r0, ROWS_T)],
                    den_out.at[c, pl.ds(r0, ROWS_T)])
    pltpu.sync_copy(acc.at[pl.ds(r0, ROWS_T)],
                    acc_out.at[c, pl.ds(r0, ROWS_T)])


def _sc2():
    f32 = jnp.float32
    mesh = plsc.VectorSubcoreMesh(core_axis_name="c", subcore_axis_name="s",
                                  num_cores=NC, num_subcores=NS)
    return pl.kernel(
        _sc2_body,
        out_type=[
            jax.ShapeDtypeStruct((NC, NP, L), f32),
            jax.ShapeDtypeStruct((NC, NP, HID), f32),
        ],
        mesh=mesh,
        scratch_types=[
            pltpu.VMEM((WIN,), jnp.int32),
            pltpu.VMEM((WIN,), jnp.int32),
            pltpu.VMEM((WIN, L), f32),
            pltpu.VMEM((WIN, L), f32),
            pltpu.VMEM((WIN, L), f32),
            pltpu.VMEM((WIN, FCH), f32),
            pltpu.VMEM((WIN, HID), f32),
            pltpu.VMEM_SHARED((NP, L), f32),
            pltpu.VMEM_SHARED((NP, L), f32),
            pltpu.VMEM_SHARED((NP, L), f32),
            pltpu.VMEM_SHARED((NP, HID), f32),
        ],
    )


def _t1_body(x_ref, w_ref, as_ref, ad_ref,
             h0_ref, h1_ref, h2_ref, h3_ref, als_ref, ald_ref):
    h = jnp.dot(x_ref[...], w_ref[...], preferred_element_type=jnp.float32)
    h0_ref[...] = h[:, 0 * FCH:1 * FCH]
    h1_ref[...] = h[:, 1 * FCH:2 * FCH]
    h2_ref[...] = h[:, 2 * FCH:3 * FCH]
    h3_ref[...] = h[:, 3 * FCH:4 * FCH]
    als_ref[...] = jnp.dot(h, as_ref[...], preferred_element_type=jnp.float32)
    ald_ref[...] = jnp.dot(h, ad_ref[...], preferred_element_type=jnp.float32)


def _t1(x, W1, A_s, A_d):
    n, d = x.shape
    blk = 1024
    grid = n // blk
    return pl.pallas_call(
        _t1_body,
        grid=(grid,),
        in_specs=[
            pl.BlockSpec((blk, d), lambda i: (i, 0)),
            pl.BlockSpec((d, DH), lambda i: (0, 0)),
            pl.BlockSpec((DH, L), lambda i: (0, 0)),
            pl.BlockSpec((DH, L), lambda i: (0, 0)),
        ],
        out_specs=[pl.BlockSpec((blk, FCH), lambda i: (i, 0))] * NCH
        + [pl.BlockSpec((blk, L), lambda i: (i, 0))] * 2,
        out_shape=[jax.ShapeDtypeStruct((n, FCH), jnp.float32)] * NCH
        + [jax.ShapeDtypeStruct((n, L), jnp.float32)] * 2,
    )(x, W1, A_s, A_d)


def _t2_body(acc_ref, den_ref, b1_ref, w2_ref, a2s_ref, a2d_ref,
             h2_ref, t2s_ref, t2d_ref):
    r = 1.0 / (den_ref[0, :, :HEADS] + den_ref[1, :, :HEADS] + 1e-16)
    heads_of_col = lax.broadcasted_iota(jnp.int32, (HEADS, DH), 1) // HID
    e8 = (lax.broadcasted_iota(jnp.int32, (HEADS, DH), 0)
          == heads_of_col).astype(jnp.float32)
    rexp = jnp.dot(r, e8, preferred_element_type=jnp.float32)
    acc = acc_ref[0] + acc_ref[1]
    h1n = jnp.maximum(acc * rexp + b1_ref[...], 0.0)
    h2 = jnp.dot(h1n, w2_ref[...], preferred_element_type=jnp.float32)
    h2_ref[...] = jnp.concatenate([h2, jnp.zeros_like(h2)], axis=1)
    t2s_ref[...] = jnp.dot(h2, a2s_ref[...], preferred_element_type=jnp.float32)
    t2d_ref[...] = jnp.dot(h2, a2d_ref[...], preferred_element_type=jnp.float32)


def _t2(acc1, den1, b1, W2, A2s, A2d):
    blk = 1024
    grid = NP // blk
    return pl.pallas_call(
        _t2_body,
        grid=(grid,),
        in_specs=[
            pl.BlockSpec((NC, blk, DH), lambda i: (0, i, 0)),
            pl.BlockSpec((NC, blk, L), lambda i: (0, i, 0)),
            pl.BlockSpec((1, DH), lambda i: (0, 0)),
            pl.BlockSpec((DH, HID), lambda i: (0, 0)),
            pl.BlockSpec((HID, L), lambda i: (0, 0)),
            pl.BlockSpec((HID, L), lambda i: (0, 0)),
        ],
        out_specs=[
            pl.BlockSpec((blk, FCH), lambda i: (i, 0)),
            pl.BlockSpec((blk, L), lambda i: (i, 0)),
            pl.BlockSpec((blk, L), lambda i: (i, 0)),
        ],
        out_shape=[
            jax.ShapeDtypeStruct((NP, FCH), jnp.float32),
            jax.ShapeDtypeStruct((NP, L), jnp.float32),
            jax.ShapeDtypeStruct((NP, L), jnp.float32),
        ],
    )(acc1, den1, b1, W2, A2s, A2d)


def _t3_body(acc_ref, den_ref, b2_ref, batch_ref, fcw_ref, fcb_ref, out_ref):
    n = acc_ref.shape[1]
    r = 1.0 / (den_ref[0, :, 0:1] + den_ref[1, :, 0:1] + 1e-16)
    h2f = (acc_ref[0] + acc_ref[1]) * r + b2_ref[...]
    g = lax.broadcasted_iota(jnp.int32, (GROUPS, n), 0)
    onehot = (batch_ref[...] == g).astype(jnp.float32)
    pooled = jnp.dot(onehot, h2f, preferred_element_type=jnp.float32)
    logits = jnp.dot(pooled, fcw_ref[...],
                     preferred_element_type=jnp.float32) + fcb_ref[...]
    m = jnp.max(logits, axis=1, keepdims=True)
    sh = logits - m
    out_ref[...] = sh - jnp.log(jnp.sum(jnp.exp(sh), axis=1, keepdims=True))


def _t3(acc2, den2, b2, batch2d, fc_w, fc_b):
    c = fc_w.shape[1]
    return pl.pallas_call(
        _t3_body,
        out_shape=jax.ShapeDtypeStruct((GROUPS, c), jnp.float32),
    )(acc2, den2, b2, batch2d, fc_w, fc_b)


def kernel(x, edge_index, batch, W1, a_src1, a_dst1, b1, W2, a_src2, a_dst2,
           b2, fc_w, fc_b):
    f32 = jnp.float32
    src = jnp.pad(edge_index[0], (0, EP - N_EDGES), constant_values=N_NODES)
    dst = jnp.pad(edge_index[1], (0, EP - N_EDGES), constant_values=N_NODES)

    eye8 = jnp.eye(HEADS, dtype=f32)
    A_s = jnp.pad((a_src1[:, :, None] * eye8[:, None, :]).reshape(DH, HEADS),
                  ((0, 0), (0, L - HEADS)))
    A_d = jnp.pad((a_dst1[:, :, None] * eye8[:, None, :]).reshape(DH, HEADS),
                  ((0, 0), (0, L - HEADS)))
    A2s = jnp.tile(a_src2.T, (1, L))
    A2d = jnp.tile(a_dst2.T, (1, L))

    x_pad = jnp.pad(x, ((0, NP - N_NODES), (0, 0)))
    h0, h1, h2c, h3, tab_s, tab_d = _t1(x_pad, W1, A_s, A_d)

    z16 = jnp.zeros((NP, L), f32)
    z128 = jnp.zeros((NP, FCH), f32)
    z64 = jnp.zeros((NP, HID), f32)

    den1, acc1 = _sc1()(tab_s, tab_d, src, dst, z16, z128, h0, h1, h2c, h3)

    h2, tab2s, tab2d = _t2(acc1, den1, b1[None, :], W2, A2s, A2d)

    den2, acc2 = _sc2()(tab2s, tab2d, src, dst, z16, z64, h2)

    batch_pad = jnp.pad(batch, (0, NP - N_NODES),
                        constant_values=GROUPS)[None, :]
    return _t3(acc2, den2, b2[None, :], batch_pad, fc_w, fc_b[None, :])


# speedup vs baseline: 10.8120x; 1.1242x over previous
import jax
import jax.numpy as jnp
from jax import lax
from jax.experimental import pallas as pl
from jax.experimental.pallas import tpu as pltpu
import jax.experimental.pallas.tpu_sc as plsc

N_NODES = 10000
NP = 10240
N_EDGES = 160000
HEADS = 8
HID = 64
DH = HEADS * HID
GROUPS = 64
L = 16

NC = 2
NS = 16
NW = NC * NS
WIN = 40
EP = 161280
EW = EP // NW
NWIN = EW // WIN
SPLITS = ((0, 24), (24, 16))
RB = 24
ROWS_T = NP // NS
FCH = 128
NCH = DH // FCH


def _sc1_body(tab_s, tab_d, src, dst, z16, z128, h0, h1, h2, h3,
              den_out, acc_out,
              srcw, dstw, rows_s, rows_d, rows,
              tabs_sh, tabd_sh, den_acc, acc):
    c = lax.axis_index("c")
    s = lax.axis_index("s")
    w = c * NS + s
    ebase = w * EW
    r0 = s * ROWS_T

    pltpu.sync_copy(tab_s.at[pl.ds(r0, ROWS_T)], tabs_sh.at[pl.ds(r0, ROWS_T)])
    pltpu.sync_copy(tab_d.at[pl.ds(r0, ROWS_T)], tabd_sh.at[pl.ds(r0, ROWS_T)])
    pltpu.sync_copy(z16.at[pl.ds(r0, ROWS_T)], den_acc.at[pl.ds(r0, ROWS_T)])
    plsc.subcore_barrier()

    for ch, tab in enumerate((h0, h1, h2, h3)):
        pltpu.sync_copy(z128.at[pl.ds(r0, ROWS_T)], acc.at[pl.ds(r0, ROWS_T)])
        plsc.subcore_barrier()

        def p2(j, carry):
            eo = pl.multiple_of(j * WIN, 8)
            pltpu.sync_copy(src.at[pl.ds(ebase + eo, WIN)], srcw)
            pltpu.sync_copy(dst.at[pl.ds(ebase + eo, WIN)], dstw)
            pltpu.sync_copy(tabs_sh.at[srcw], rows_s)
            pltpu.sync_copy(tabd_sh.at[dstw], rows_d)
            for o, m in SPLITS:
                pltpu.sync_copy(tab.at[srcw.at[pl.ds(o, m)]],
                                rows.at[pl.ds(0, m)])
                for e in range(m):
                    a = rows_s[o + e, :] + rows_d[o + e, :]
                    a = jnp.maximum(a, 0.2 * a)
                    v = jnp.exp(a)
                    if ch == 0:
                        rows_s[o + e, :] = v
                    s0 = v[2 * ch]
                    s1 = v[2 * ch + 1]
                    for k in range(4):
                        rows[e, pl.ds(k * L, L)] = (
                            rows[e, pl.ds(k * L, L)] * s0)
                    for k in range(4, 8):
                        rows[e, pl.ds(k * L, L)] = (
                            rows[e, pl.ds(k * L, L)] * s1)
                pltpu.sync_copy(rows.at[pl.ds(0, m)],
                                acc.at[dstw.at[pl.ds(o, m)]], add=True)
            if ch == 0:
                pltpu.sync_copy(rows_s, den_acc.at[dstw], add=True)
            return carry

        lax.fori_loop(0, NWIN, p2, 0)
        plsc.subcore_barrier()
        if ch == 0:
            pltpu.sync_copy(den_acc.at[pl.ds(r0, ROWS_T)],
                            den_out.at[c, pl.ds(r0, ROWS_T)])
        pltpu.sync_copy(acc.at[pl.ds(r0, ROWS_T)],
                        acc_out.at[c, pl.ds(r0, ROWS_T), pl.ds(ch * FCH, FCH)])
        plsc.subcore_barrier()


def _sc1():
    f32 = jnp.float32
    mesh = plsc.VectorSubcoreMesh(core_axis_name="c", subcore_axis_name="s",
                                  num_cores=NC, num_subcores=NS)
    return pl.kernel(
        _sc1_body,
        out_type=[
            jax.ShapeDtypeStruct((NC, NP, L), f32),
            jax.ShapeDtypeStruct((NC, NP, DH), f32),
        ],
        mesh=mesh,
        scratch_types=[
            pltpu.VMEM((WIN,), jnp.int32),
            pltpu.VMEM((WIN,), jnp.int32),
            pltpu.VMEM((WIN, L), f32),
            pltpu.VMEM((WIN, L), f32),
            pltpu.VMEM((RB, FCH), f32),
            pltpu.VMEM_SHARED((NP, L), f32),
            pltpu.VMEM_SHARED((NP, L), f32),
            pltpu.VMEM_SHARED((NP, L), f32),
            pltpu.VMEM_SHARED((NP, FCH), f32),
        ],
    )


def _sc2_body(tab_s, tab_d, src, dst, z16, z64, hfeat,
              den_out, acc_out,
              srcw, dstw, rows_s, rows_d, rows, msg,
              tabs_sh, tabd_sh, den_acc, acc):
    c = lax.axis_index("c")
    s = lax.axis_index("s")
    w = c * NS + s
    ebase = w * EW
    r0 = s * ROWS_T

    pltpu.sync_copy(tab_s.at[pl.ds(r0, ROWS_T)], tabs_sh.at[pl.ds(r0, ROWS_T)])
    pltpu.sync_copy(tab_d.at[pl.ds(r0, ROWS_T)], tabd_sh.at[pl.ds(r0, ROWS_T)])
    pltpu.sync_copy(z16.at[pl.ds(r0, ROWS_T)], den_acc.at[pl.ds(r0, ROWS_T)])
    pltpu.sync_copy(z64.at[pl.ds(r0, ROWS_T)], acc.at[pl.ds(r0, ROWS_T)])
    plsc.subcore_barrier()

    def p(j, carry):
        eo = pl.multiple_of(j * WIN, 8)
        pltpu.sync_copy(src.at[pl.ds(ebase + eo, WIN)], srcw)
        pltpu.sync_copy(dst.at[pl.ds(ebase + eo, WIN)], dstw)
        pltpu.sync_copy(tabs_sh.at[srcw], rows_s)
        pltpu.sync_copy(tabd_sh.at[dstw], rows_d)
        for o, m in SPLITS:
            pltpu.sync_copy(hfeat.at[srcw.at[pl.ds(o, m)]],
                            rows.at[pl.ds(0, m)])
            for e in range(m):
                a = rows_s[o + e, :] + rows_d[o + e, :]
                a = jnp.maximum(a, 0.2 * a)
                v = jnp.exp(a)
                rows_s[o + e, :] = v
                s0 = v[0]
                for k in range(4):
                    msg[e, pl.ds(k * L, L)] = rows[e, pl.ds(k * L, L)] * s0
            pltpu.sync_copy(msg.at[pl.ds(0, m)],
                            acc.at[dstw.at[pl.ds(o, m)]], add=True)
        pltpu.sync_copy(rows_s, den_acc.at[dstw], add=True)
        return carry

    lax.fori_loop(0, NWIN, p, 0)
    plsc.subcore_barrier()
    pltpu.sync_copy(den_acc.at[pl.ds(r0, ROWS_T)],
                    den_out.at[c, pl.ds(r0, ROWS_T)])
    pltpu.sync_copy(acc.at[pl.ds(r0, ROWS_T)],
                    acc_out.at[c, pl.ds(r0, ROWS_T)])


def _sc2():
    f32 = jnp.float32
    mesh = plsc.VectorSubcoreMesh(core_axis_name="c", subcore_axis_name="s",
                                  num_cores=NC, num_subcores=NS)
    return pl.kernel(
        _sc2_body,
        out_type=[
            jax.ShapeDtypeStruct((NC, NP, L), f32),
            jax.ShapeDtypeStruct((NC, NP, HID), f32),
        ],
        mesh=mesh,
        scratch_types=[
            pltpu.VMEM((WIN,), jnp.int32),
            pltpu.VMEM((WIN,), jnp.int32),
            pltpu.VMEM((WIN, L), f32),
            pltpu.VMEM((WIN, L), f32),
            pltpu.VMEM((RB, FCH), f32),
            pltpu.VMEM((RB, HID), f32),
            pltpu.VMEM_SHARED((NP, L), f32),
            pltpu.VMEM_SHARED((NP, L), f32),
            pltpu.VMEM_SHARED((NP, L), f32),
            pltpu.VMEM_SHARED((NP, HID), f32),
        ],
    )


def _t1_body(x_ref, w_ref, as_ref, ad_ref,
             h0_ref, h1_ref, h2_ref, h3_ref, als_ref, ald_ref):
    h = jnp.dot(x_ref[...], w_ref[...], preferred_element_type=jnp.float32)
    h0_ref[...] = h[:, 0 * FCH:1 * FCH]
    h1_ref[...] = h[:, 1 * FCH:2 * FCH]
    h2_ref[...] = h[:, 2 * FCH:3 * FCH]
    h3_ref[...] = h[:, 3 * FCH:4 * FCH]
    als_ref[...] = jnp.dot(h, as_ref[...], preferred_element_type=jnp.float32)
    ald_ref[...] = jnp.dot(h, ad_ref[...], preferred_element_type=jnp.float32)


def _t1(x, W1, A_s, A_d):
    n, d = x.shape
    blk = 1024
    grid = n // blk
    return pl.pallas_call(
        _t1_body,
        grid=(grid,),
        in_specs=[
            pl.BlockSpec((blk, d), lambda i: (i, 0)),
            pl.BlockSpec((d, DH), lambda i: (0, 0)),
            pl.BlockSpec((DH, L), lambda i: (0, 0)),
            pl.BlockSpec((DH, L), lambda i: (0, 0)),
        ],
        out_specs=[pl.BlockSpec((blk, FCH), lambda i: (i, 0))] * NCH
        + [pl.BlockSpec((blk, L), lambda i: (i, 0))] * 2,
        out_shape=[jax.ShapeDtypeStruct((n, FCH), jnp.float32)] * NCH
        + [jax.ShapeDtypeStruct((n, L), jnp.float32)] * 2,
    )(x, W1, A_s, A_d)


def _t2_body(acc_ref, den_ref, b1_ref, w2_ref, a2s_ref, a2d_ref,
             h2_ref, t2s_ref, t2d_ref):
    r = 1.0 / (den_ref[0, :, :HEADS] + den_ref[1, :, :HEADS] + 1e-16)
    heads_of_col = lax.broadcasted_iota(jnp.int32, (HEADS, DH), 1) // HID
    e8 = (lax.broadcasted_iota(jnp.int32, (HEADS, DH), 0)
          == heads_of_col).astype(jnp.float32)
    rexp = jnp.dot(r, e8, preferred_element_type=jnp.float32)
    acc = acc_ref[0] + acc_ref[1]
    h1n = jnp.maximum(acc * rexp + b1_ref[...], 0.0)
    h2 = jnp.dot(h1n, w2_ref[...], preferred_element_type=jnp.float32)
    h2_ref[...] = jnp.concatenate([h2, jnp.zeros_like(h2)], axis=1)
    t2s_ref[...] = jnp.dot(h2, a2s_ref[...], preferred_element_type=jnp.float32)
    t2d_ref[...] = jnp.dot(h2, a2d_ref[...], preferred_element_type=jnp.float32)


def _t2(acc1, den1, b1, W2, A2s, A2d):
    blk = 1024
    grid = NP // blk
    return pl.pallas_call(
        _t2_body,
        grid=(grid,),
        in_specs=[
            pl.BlockSpec((NC, blk, DH), lambda i: (0, i, 0)),
            pl.BlockSpec((NC, blk, L), lambda i: (0, i, 0)),
            pl.BlockSpec((1, DH), lambda i: (0, 0)),
            pl.BlockSpec((DH, HID), lambda i: (0, 0)),
            pl.BlockSpec((HID, L), lambda i: (0, 0)),
            pl.BlockSpec((HID, L), lambda i: (0, 0)),
        ],
        out_specs=[
            pl.BlockSpec((blk, FCH), lambda i: (i, 0)),
            pl.BlockSpec((blk, L), lambda i: (i, 0)),
            pl.BlockSpec((blk, L), lambda i: (i, 0)),
        ],
        out_shape=[
            jax.ShapeDtypeStruct((NP, FCH), jnp.float32),
            jax.ShapeDtypeStruct((NP, L), jnp.float32),
            jax.ShapeDtypeStruct((NP, L), jnp.float32),
        ],
    )(acc1, den1, b1, W2, A2s, A2d)


def _t3_body(acc_ref, den_ref, b2_ref, batch_ref, fcw_ref, fcb_ref, out_ref):
    n = acc_ref.shape[1]
    r = 1.0 / (den_ref[0, :, 0:1] + den_ref[1, :, 0:1] + 1e-16)
    h2f = (acc_ref[0] + acc_ref[1]) * r + b2_ref[...]
    g = lax.broadcasted_iota(jnp.int32, (GROUPS, n), 0)
    onehot = (batch_ref[...] == g).astype(jnp.float32)
    pooled = jnp.dot(onehot, h2f, preferred_element_type=jnp.float32)
    logits = jnp.dot(pooled, fcw_ref[...],
                     preferred_element_type=jnp.float32) + fcb_ref[...]
    m = jnp.max(logits, axis=1, keepdims=True)
    sh = logits - m
    out_ref[...] = sh - jnp.log(jnp.sum(jnp.exp(sh), axis=1, keepdims=True))


def _t3(acc2, den2, b2, batch2d, fc_w, fc_b):
    c = fc_w.shape[1]
    return pl.pallas_call(
        _t3_body,
        out_shape=jax.ShapeDtypeStruct((GROUPS, c), jnp.float32),
    )(acc2, den2, b2, batch2d, fc_w, fc_b)


def kernel(x, edge_index, batch, W1, a_src1, a_dst1, b1, W2, a_src2, a_dst2,
           b2, fc_w, fc_b):
    f32 = jnp.float32
    src = jnp.pad(edge_index[0], (0, EP - N_EDGES), constant_values=N_NODES)
    dst = jnp.pad(edge_index[1], (0, EP - N_EDGES), constant_values=N_NODES)

    eye8 = jnp.eye(HEADS, dtype=f32)
    A_s = jnp.pad((a_src1[:, :, None] * eye8[:, None, :]).reshape(DH, HEADS),
                  ((0, 0), (0, L - HEADS)))
    A_d = jnp.pad((a_dst1[:, :, None] * eye8[:, None, :]).reshape(DH, HEADS),
                  ((0, 0), (0, L - HEADS)))
    A2s = jnp.tile(a_src2.T, (1, L))
    A2d = jnp.tile(a_dst2.T, (1, L))

    x_pad = jnp.pad(x, ((0, NP - N_NODES), (0, 0)))
    h0, h1, h2c, h3, tab_s, tab_d = _t1(x_pad, W1, A_s, A_d)

    z16 = jnp.zeros((NP, L), f32)
    z128 = jnp.zeros((NP, FCH), f32)
    z64 = jnp.zeros((NP, HID), f32)

    den1, acc1 = _sc1()(tab_s, tab_d, src, dst, z16, z128, h0, h1, h2c, h3)

    h2, tab2s, tab2d = _t2(acc1, den1, b1[None, :], W2, A2s, A2d)

    den2, acc2 = _sc2()(tab2s, tab2d, src, dst, z16, z64, h2)

    batch_pad = jnp.pad(batch, (0, NP - N_NODES),
                        constant_values=GROUPS)[None, :]
    return _t3(acc2, den2, b2[None, :], batch_pad, fc_w, fc_b[None, :])
